# baseline (speedup 1.0000x reference)
import jax
import jax.numpy as jnp
from jax.experimental import pallas as pl
from jax.experimental.pallas import tpu as pltpu

_BR = 2048
_N_ROWS = 16384
_N_CLASSES = 1000
_N_LABELS = 10
_LANES = 128
_N_BLOCKS = 8
_N_STEPS = _N_ROWS // _BR


def _dense_kernel(hbm_ref, lab_ref, p_ref, b0, b1, s0, s1):
    bufs = (b0, b1)
    sems = (s0, s1)

    def copy(i):
        return pltpu.make_async_copy(
            hbm_ref.at[pl.ds(i * _BR, _BR), :], bufs[i % 2], sems[i % 2])

    copy(0).start()
    for i in range(_N_STEPS):
        if i + 1 < _N_STEPS:
            copy(i + 1).start()
        copy(i).wait()
        buf = bufs[i % 2]
        rs = pl.ds(i * _BR, _BR)
        labs = lab_ref[rs, :]
        hi = labs >> 7
        lo = labs & (_LANES - 1)
        g = jnp.zeros((_BR, _N_LABELS), jnp.float32)
        for b in range(_N_BLOCKS):
            width = min(_LANES, _N_CLASSES - b * _LANES)
            idx = lo if width == _LANES else jnp.minimum(lo, width - 1)
            cand = jnp.take_along_axis(buf[:, b * _LANES:b * _LANES + width],
                                       idx, axis=1)
            g = jnp.where(hi == b, cand, g)
        den = jnp.sum(jnp.exp(buf[...]), axis=1, keepdims=True)
        p_ref[rs, :] = jnp.exp(g) / den


def _reduce_kernel(labt_ref, p_ref, acc_ref):
    xt = labt_ref[...]
    p = p_ref[...]
    rows = [xt[k:k + 1, :] for k in range(_N_LABELS)]
    keeps = []
    for k in range(_N_LABELS):
        keep = rows[k] != -1
        for j in range(k):
            keep = keep & (rows[j] != rows[k])
        keeps.append(jnp.where(keep, 1.0, 0.0))
    keep_t = jnp.concatenate(keeps, axis=0)
    m = jax.lax.dot_general(keep_t, p, (((1,), (0,)), ((), ())),
                            preferred_element_type=jnp.float32)
    r = jax.lax.broadcasted_iota(jnp.int32, (_N_LABELS, _N_LABELS), 0)
    c = jax.lax.broadcasted_iota(jnp.int32, (_N_LABELS, _N_LABELS), 1)
    total = jnp.sum(jnp.where(r == c, m, 0.0))
    acc_ref[...] = total.reshape(1, 1) * (1.0 / _N_ROWS)


def kernel(outputs, complementary_labels):
    labels_t = complementary_labels.T

    p = pl.pallas_call(
        _dense_kernel,
        in_specs=[
            pl.BlockSpec(memory_space=pl.ANY),
            pl.BlockSpec((_N_ROWS, _N_LABELS), lambda: (0, 0)),
        ],
        out_specs=pl.BlockSpec((_N_ROWS, _N_LABELS), lambda: (0, 0)),
        out_shape=jax.ShapeDtypeStruct((_N_ROWS, _N_LABELS), jnp.float32),
        scratch_shapes=[
            pltpu.VMEM((_BR, _N_CLASSES), jnp.float32),
            pltpu.VMEM((_BR, _N_CLASSES), jnp.float32),
            pltpu.SemaphoreType.DMA,
            pltpu.SemaphoreType.DMA,
        ],
    )(outputs, complementary_labels)

    acc = pl.pallas_call(
        _reduce_kernel,
        in_specs=[
            pl.BlockSpec((_N_LABELS, _N_ROWS), lambda: (0, 0)),
            pl.BlockSpec((_N_ROWS, _N_LABELS), lambda: (0, 0)),
        ],
        out_specs=pl.BlockSpec((1, 1), lambda: (0, 0)),
        out_shape=jax.ShapeDtypeStruct((1, 1), jnp.float32),
    )(labels_t, p)
    return acc[0, 0]
